# baseline (speedup 1.0000x reference)
import functools

import jax
import jax.numpy as jnp
from jax import lax
from jax.experimental import pallas as pl
from jax.experimental.pallas import tpu as pltpu
from jax.experimental.pallas import tpu_sc as plsc

BATCH = 16384
D = 64
NW = 32
BPW = BATCH // NW
K = 16
TBLK = 2048

NU = 1000000
CHUNK_U = 512
NCHUNKS = (NU + CHUNK_U - 1) // CHUNK_U
CPW = (NCHUNKS - (176128 // CHUNK_U) + NW - 1) // NW
TAILSTART = (NU // 128) * 128
SLABMAX = ((NU - CHUNK_U) // 128) * 128
RING = 16
HEAD = 176128
HCHUNK = HEAD // CHUNK_U


def _tc_transpose(tabT, n_rows):
    grid = (n_rows + TBLK - 1) // TBLK

    def body(x_ref, o_ref):
        row = lax.broadcasted_iota(jnp.int32, (D, D), 0)
        col = lax.broadcasted_iota(jnp.int32, (D, D), 1)
        eye = jnp.where(row == col, 1.0, 0.0).astype(jnp.float32)
        o_ref[:] = jax.lax.dot_general(
            x_ref[:], eye, (((0,), (0,)), ((), ())),
            preferred_element_type=jnp.float32)

    return pl.pallas_call(
        body,
        grid=(grid,),
        in_specs=[pl.BlockSpec((D, TBLK), lambda i: (0, i))],
        out_specs=pl.BlockSpec((TBLK, D), lambda i: (i, 0)),
        out_shape=jax.ShapeDtypeStruct((n_rows, D), jnp.float32),
    )(tabT)


def _lane_extract(vec16, lane):
    lanes = lax.iota(jnp.int32, 16)
    return jnp.sum(jnp.where(lanes == lane, vec16, jnp.zeros_like(vec16)))


def _sc_user_stream(su, pu, utabT, utail):
    mesh = plsc.VectorSubcoreMesh(core_axis_name="c", subcore_axis_name="s")
    ntail = NU - TAILSTART

    @functools.partial(
        pl.kernel,
        mesh=mesh,
        out_type=jax.ShapeDtypeStruct((BATCH, D), jnp.float32),
        scratch_types=[
            pltpu.VMEM((BATCH,), jnp.int32),
            pltpu.VMEM((BATCH,), jnp.int32),
            pltpu.VMEM((2, 8, 8, CHUNK_U), jnp.float32),
            pltpu.VMEM((ntail, D), jnp.float32),
            pltpu.VMEM((RING, D), jnp.float32),
            pltpu.SemaphoreType.DMA((2,)),
            pltpu.SemaphoreType.DMA((RING,)),
            pltpu.SemaphoreType.DMA,
        ],
        compiler_params=pltpu.CompilerParams(needs_layout_passes=False),
    )
    def body(su_hbm, pu_hbm, utab_hbm, utail_hbm, ug_hbm,
             su_v, pu_v, slab_v, utail_v, ring_v, bsem, wsem, ssem):
        cid = lax.axis_index("c")
        sid = lax.axis_index("s")
        wid = sid * 2 + cid
        c_lo = HCHUNK + wid * CPW
        c_hi = jnp.minimum(c_lo + CPW, NCHUNKS)

        pltpu.sync_copy(su_hbm, su_v)
        pltpu.sync_copy(pu_hbm, pu_v)
        pltpu.sync_copy(utail_hbm, utail_v)

        lo_user = c_lo * CHUNK_U

        def cnt(i, acc):
            v = su_v[pl.ds(i * 16, 16)]
            ones = jnp.where(v < lo_user, 1, 0)
            return acc + jnp.sum(ones)

        cur0 = lax.fori_loop(0, BATCH // 16, cnt, jnp.int32(0))

        def slab_start(c):
            return jnp.minimum(c * CHUNK_U, SLABMAX)

        def issue_chunk(c, slot):
            st = slab_start(c)
            for i in range(8):
                pltpu.async_copy(
                    utab_hbm.at[pl.ds(i * 8, 8), pl.ds(st, CHUNK_U)],
                    slab_v.at[slot, i], bsem.at[slot])

        def wait_chunk(slot):
            for i in range(8):
                pltpu.make_async_copy(
                    utab_hbm.at[pl.ds(0, 8), pl.ds(0, CHUNK_U)],
                    slab_v.at[slot, i], bsem.at[slot]).wait()

        @pl.when(c_lo < c_hi)
        def _():
            issue_chunk(c_lo, 0)

        @pl.when(c_lo + 1 < c_hi)
        def _():
            issue_chunk(c_lo + 1, 1)

        lanes = lax.iota(jnp.int32, 16)
        dlo = [lax.iota(jnp.int32, 16) + 16 * g for g in range(4)]

        def chunk_body(c, carry):
            cur, grc = carry
            slot = lax.rem(c, 2)
            wait_chunk(slot)
            hi_user = (c + 1) * CHUNK_U
            st = slab_start(c)

            def cond(cs):
                cur_, _ = cs
                in_range = cur_ < BATCH
                v = su_v[pl.ds((cur_ // 16) * 16, 16)]
                u = _lane_extract(v, lax.rem(cur_, 16))
                return jnp.logical_and(in_range, u < hi_user)

            def row_body(cs):
                cur_, grc_ = cs
                sv = su_v[pl.ds((cur_ // 16) * 16, 16)]
                pv = pu_v[pl.ds((cur_ // 16) * 16, 16)]
                lane = lax.rem(cur_, 16)
                u = _lane_extract(sv, lane)
                p = _lane_extract(pv, lane)
                cc = jnp.clip(u - st, 0, CHUNK_U - 1)
                is_tail = u >= TAILSTART
                ct = jnp.clip(u - TAILSTART, 0, ntail - 1)
                ri = lax.rem(grc_, RING)

                @pl.when(grc_ >= RING)
                def _():
                    pltpu.make_async_copy(
                        ring_v.at[0], ug_hbm.at[0], wsem.at[ri]).wait()

                for g in range(4):
                    dg = dlo[g]
                    main = plsc.load_gather(
                        slab_v,
                        [jnp.full((16,), slot, jnp.int32),
                         lax.shift_right_logical(dg, 3),
                         lax.bitwise_and(dg, 7),
                         jnp.full((16,), cc, jnp.int32)])
                    tail = utail_v[ct, pl.ds(16 * g, 16)]
                    ring_v[ri, pl.ds(16 * g, 16)] = jnp.where(
                        is_tail, tail, main)
                pltpu.async_copy(ring_v.at[ri], ug_hbm.at[p], wsem.at[ri])
                return (cur_ + 1, grc_ + 1)

            cur, grc = lax.while_loop(cond, row_body, (cur, grc))

            @pl.when(c + 2 < c_hi)
            def _():
                issue_chunk(c + 2, slot)

            return (cur, grc)

        cur, grc = lax.fori_loop(c_lo, c_hi, chunk_body, (cur0, jnp.int32(0)))

        def drain(i, grc_):
            @pl.when(i < jnp.minimum(grc_, RING))
            def _():
                pltpu.make_async_copy(
                    ring_v.at[0], ug_hbm.at[0], wsem.at[i]).wait()
            return grc_

        lax.fori_loop(0, RING, drain, grc)

    return body(su, pu, utabT, utail)


def _sc_compute(user_idx, movie_idx, u_head, u_g, movie_tab, w16, b16):
    mesh = plsc.VectorSubcoreMesh(core_axis_name="c", subcore_axis_name="s")

    @functools.partial(
        pl.kernel,
        mesh=mesh,
        out_type=jax.ShapeDtypeStruct((BATCH,), jnp.float32),
        scratch_types=[
            pltpu.VMEM((BPW,), jnp.int32),
            pltpu.VMEM((BPW,), jnp.int32),
            pltpu.VMEM((BPW // 2, 2 * D), jnp.float32),
            pltpu.VMEM((BPW // 2, 2 * D), jnp.float32),
            pltpu.VMEM((4, 16), jnp.float32),
            pltpu.VMEM((16,), jnp.float32),
            pltpu.VMEM((BPW,), jnp.float32),
            pltpu.SemaphoreType.DMA,
        ],
        compiler_params=pltpu.CompilerParams(needs_layout_passes=False),
    )
    def body(uidx_hbm, midx_hbm, uhead_hbm, ug_hbm, mtab_hbm, w_hbm, b_hbm,
             out_hbm, uidx_v, midx_v, urows_v, mrows_v, w_v, b_v, out_v, sem):
        cid = lax.axis_index("c")
        sid = lax.axis_index("s")
        wid = sid * 2 + cid
        base = wid * BPW

        pltpu.sync_copy(uidx_hbm.at[pl.ds(base, BPW)], uidx_v)
        pltpu.sync_copy(midx_hbm.at[pl.ds(base, BPW)], midx_v)
        pltpu.sync_copy(w_hbm, w_v)
        pltpu.sync_copy(b_hbm, b_v)

        def burst(c, _):
            r0 = c * K
            uvec = uidx_v[pl.ds(r0, K)]
            mvec = midx_v[pl.ds(r0, K)]
            for k in range(K):
                r = r0 + k
                u = uvec[k]

                @pl.when(u < HEAD)
                def _():
                    pltpu.async_copy(
                        uhead_hbm.at[u],
                        urows_v.at[r // 2, pl.ds((r % 2) * D, D)], sem)

                @pl.when(u >= HEAD)
                def _():
                    pltpu.async_copy(
                        ug_hbm.at[base + r],
                        urows_v.at[r // 2, pl.ds((r % 2) * D, D)], sem)

                pltpu.async_copy(
                    mtab_hbm.at[mvec[k]],
                    mrows_v.at[r // 2, pl.ds((r % 2) * D, D)], sem)
            for k in range(K):
                r = r0 + k
                pltpu.make_async_copy(
                    mtab_hbm.at[0],
                    mrows_v.at[r // 2, pl.ds((r % 2) * D, D)], sem).wait()
                pltpu.make_async_copy(
                    mtab_hbm.at[0],
                    urows_v.at[r // 2, pl.ds((r % 2) * D, D)], sem).wait()
            return 0

        lax.fori_loop(0, BPW // K, burst, 0)

        w0 = w_v[0, :]
        w1 = w_v[1, :]
        w2 = w_v[2, :]
        w3 = w_v[3, :]
        bias = b_v[:]
        lanes = lax.iota(jnp.int32, 16)

        def block(blk, _):
            r0 = blk * 16
            res = jnp.zeros((16,), jnp.float32)
            for k in range(16):
                rr = r0 // 2 + k // 2
                c0 = (k % 2) * D
                t = (urows_v[rr, pl.ds(c0, 16)] * mrows_v[rr, pl.ds(c0, 16)] * w0
                     + urows_v[rr, pl.ds(c0 + 16, 16)] * mrows_v[rr, pl.ds(c0 + 16, 16)] * w1
                     + urows_v[rr, pl.ds(c0 + 32, 16)] * mrows_v[rr, pl.ds(c0 + 32, 16)] * w2
                     + urows_v[rr, pl.ds(c0 + 48, 16)] * mrows_v[rr, pl.ds(c0 + 48, 16)] * w3)
                s = jnp.sum(t + bias)
                res = jnp.where(lanes == k, s, res)
            out_v[pl.ds(r0, 16)] = res
            return 0

        lax.fori_loop(0, BPW // 16, block, 0)

        pltpu.sync_copy(out_v, out_hbm.at[pl.ds(base, BPW)])

    return body(user_idx, movie_idx, u_head, u_g, movie_tab, w16, b16)


def kernel(user_idx, movie_idx, user_table, movie_table, fc_w, fc_b):
    uidx = user_idx.astype(jnp.int32)
    midx = movie_idx.astype(jnp.int32)
    w16 = fc_w.reshape(4, 16)
    b16 = jnp.broadcast_to(fc_b.reshape(1, 1) / 16.0, (1, 16)).reshape(16)

    perm = jnp.argsort(uidx)
    su = jnp.take(uidx, perm)
    pu = perm.astype(jnp.int32)
    utail = user_table[TAILSTART:, :]

    movie_rm = _tc_transpose(movie_table.T, movie_table.shape[0])
    user_head = _tc_transpose(user_table.T, HEAD)
    u_g = _sc_user_stream(su, pu, user_table.T, utail)
    out = _sc_compute(uidx, midx, user_head, u_g, movie_rm, w16, b16)
    return out


# speedup vs baseline: 1.2326x; 1.2326x over previous
import functools

import jax
import jax.numpy as jnp
from jax import lax
from jax.experimental import pallas as pl
from jax.experimental.pallas import tpu as pltpu
from jax.experimental.pallas import tpu_sc as plsc

BATCH = 16384
D = 64
NW = 32
BPW = BATCH // NW
K = 16
TBLK = 2048

NU = 1000000
CHUNK_U = 512
NCHUNKS = (NU + CHUNK_U - 1) // CHUNK_U
CPW = (NCHUNKS + NW - 1) // NW
TAILSTART = (NU // 128) * 128
SLABMAX = ((NU - CHUNK_U) // 128) * 128
RING = 16


def _tc_transpose(tabT, n_rows):
    grid = (n_rows + TBLK - 1) // TBLK

    def body(x_ref, o_ref):
        row = lax.broadcasted_iota(jnp.int32, (D, D), 0)
        col = lax.broadcasted_iota(jnp.int32, (D, D), 1)
        eye = jnp.where(row == col, 1.0, 0.0).astype(jnp.float32)
        o_ref[:] = jax.lax.dot_general(
            x_ref[:], eye, (((0,), (0,)), ((), ())),
            preferred_element_type=jnp.float32)

    return pl.pallas_call(
        body,
        grid=(grid,),
        in_specs=[pl.BlockSpec((D, TBLK), lambda i: (0, i))],
        out_specs=pl.BlockSpec((TBLK, D), lambda i: (i, 0)),
        out_shape=jax.ShapeDtypeStruct((n_rows, D), jnp.float32),
    )(tabT)


def _lane_extract(vec16, lane):
    lanes = lax.iota(jnp.int32, 16)
    return jnp.sum(jnp.where(lanes == lane, vec16, jnp.zeros_like(vec16)))


def _sc_user_stream(su, pu, utabT, utail):
    mesh = plsc.VectorSubcoreMesh(core_axis_name="c", subcore_axis_name="s")
    ntail = NU - TAILSTART

    @functools.partial(
        pl.kernel,
        mesh=mesh,
        out_type=jax.ShapeDtypeStruct((BATCH, D), jnp.float32),
        scratch_types=[
            pltpu.VMEM((BATCH,), jnp.int32),
            pltpu.VMEM((BATCH,), jnp.int32),
            pltpu.VMEM((2, D, CHUNK_U), jnp.float32),
            pltpu.VMEM((ntail, D), jnp.float32),
            pltpu.VMEM((RING, D), jnp.float32),
            pltpu.SemaphoreType.DMA((2,)),
            pltpu.SemaphoreType.DMA((RING,)),
            pltpu.SemaphoreType.DMA,
        ],
        compiler_params=pltpu.CompilerParams(needs_layout_passes=False),
    )
    def body(su_hbm, pu_hbm, utab_hbm, utail_hbm, ug_hbm,
             su_v, pu_v, slab_v, utail_v, ring_v, bsem, wsem, ssem):
        cid = lax.axis_index("c")
        sid = lax.axis_index("s")
        wid = sid * 2 + cid
        c_lo = wid * CPW
        c_hi = jnp.minimum(c_lo + CPW, NCHUNKS)

        pltpu.sync_copy(su_hbm, su_v)
        pltpu.sync_copy(pu_hbm, pu_v)
        pltpu.sync_copy(utail_hbm, utail_v)

        lo_user = c_lo * CHUNK_U

        def cnt(i, acc):
            v = su_v[pl.ds(i * 16, 16)]
            ones = jnp.where(v < lo_user, 1, 0)
            return acc + jnp.sum(ones)

        cur0 = lax.fori_loop(0, BATCH // 16, cnt, jnp.int32(0))

        def slab_start(c):
            return jnp.minimum(c * CHUNK_U, SLABMAX)

        def issue_chunk(c, slot):
            st = slab_start(c)
            pltpu.async_copy(
                utab_hbm.at[:, pl.ds(st, CHUNK_U)],
                slab_v.at[slot], bsem.at[slot])

        def wait_chunk(slot):
            pltpu.make_async_copy(
                utab_hbm.at[:, pl.ds(0, CHUNK_U)],
                slab_v.at[slot], bsem.at[slot]).wait()

        @pl.when(c_lo < c_hi)
        def _():
            issue_chunk(c_lo, 0)

        @pl.when(c_lo + 1 < c_hi)
        def _():
            issue_chunk(c_lo + 1, 1)

        lanes = lax.iota(jnp.int32, 16)
        dlo = [lax.iota(jnp.int32, 16) + 16 * g for g in range(4)]

        def chunk_body(c, carry):
            cur, grc = carry
            slot = lax.rem(c, 2)
            wait_chunk(slot)
            hi_user = (c + 1) * CHUNK_U
            st = slab_start(c)

            def cond(cs):
                cur_, _ = cs
                in_range = cur_ < BATCH
                v = su_v[pl.ds((cur_ // 16) * 16, 16)]
                u = _lane_extract(v, lax.rem(cur_, 16))
                return jnp.logical_and(in_range, u < hi_user)

            def row_body(cs):
                cur_, grc_ = cs
                sv = su_v[pl.ds((cur_ // 16) * 16, 16)]
                pv = pu_v[pl.ds((cur_ // 16) * 16, 16)]
                lane = lax.rem(cur_, 16)
                u = _lane_extract(sv, lane)
                p = _lane_extract(pv, lane)
                cc = jnp.clip(u - st, 0, CHUNK_U - 1)
                is_tail = u >= TAILSTART
                ct = jnp.clip(u - TAILSTART, 0, ntail - 1)
                ri = lax.rem(grc_, RING)

                @pl.when(grc_ >= RING)
                def _():
                    pltpu.make_async_copy(
                        ring_v.at[0], ug_hbm.at[0], wsem.at[ri]).wait()

                for g in range(4):
                    dg = dlo[g]
                    main = plsc.load_gather(
                        slab_v,
                        [jnp.full((16,), slot, jnp.int32),
                         dg,
                         jnp.full((16,), cc, jnp.int32)])
                    tail = utail_v[ct, pl.ds(16 * g, 16)]
                    ring_v[ri, pl.ds(16 * g, 16)] = jnp.where(
                        is_tail, tail, main)
                pltpu.async_copy(ring_v.at[ri], ug_hbm.at[p], wsem.at[ri])
                return (cur_ + 1, grc_ + 1)

            cur, grc = lax.while_loop(cond, row_body, (cur, grc))

            @pl.when(c + 2 < c_hi)
            def _():
                issue_chunk(c + 2, slot)

            return (cur, grc)

        cur, grc = lax.fori_loop(c_lo, c_hi, chunk_body, (cur0, jnp.int32(0)))

        def drain(i, grc_):
            @pl.when(i < jnp.minimum(grc_, RING))
            def _():
                pltpu.make_async_copy(
                    ring_v.at[0], ug_hbm.at[0], wsem.at[i]).wait()
            return grc_

        lax.fori_loop(0, RING, drain, grc)

    return body(su, pu, utabT, utail)


def _sc_compute(movie_idx, u_g, movie_tab, w16, b16):
    mesh = plsc.VectorSubcoreMesh(core_axis_name="c", subcore_axis_name="s")

    @functools.partial(
        pl.kernel,
        mesh=mesh,
        out_type=jax.ShapeDtypeStruct((BATCH,), jnp.float32),
        scratch_types=[
            pltpu.VMEM((BPW,), jnp.int32),
            pltpu.VMEM((8, BPW // 8, D), jnp.float32),
            pltpu.VMEM((BPW // 2, 2 * D), jnp.float32),
            pltpu.VMEM((4, 16), jnp.float32),
            pltpu.VMEM((16,), jnp.float32),
            pltpu.VMEM((BPW,), jnp.float32),
            pltpu.SemaphoreType.DMA,
        ],
        compiler_params=pltpu.CompilerParams(needs_layout_passes=False),
    )
    def body(midx_hbm, ug_hbm, mtab_hbm, w_hbm, b_hbm, out_hbm,
             midx_v, urows_v, mrows_v, w_v, b_v, out_v, sem):
        cid = lax.axis_index("c")
        sid = lax.axis_index("s")
        wid = sid * 2 + cid
        base = wid * BPW

        pltpu.sync_copy(midx_hbm.at[pl.ds(base, BPW)], midx_v)
        pltpu.sync_copy(w_hbm, w_v)
        pltpu.sync_copy(b_hbm, b_v)

        ucopies = []
        for j in range(8):
            ucopies.append(pltpu.async_copy(
                ug_hbm.at[pl.ds(base + j * (BPW // 8), BPW // 8), :],
                urows_v.at[j], sem))

        def burst(c, _):
            r0 = c * K
            mvec = midx_v[pl.ds(r0, K)]
            copies = []
            for k in range(K):
                r = r0 + k
                copies.append(pltpu.async_copy(
                    mtab_hbm.at[mvec[k]],
                    mrows_v.at[r // 2, pl.ds((r % 2) * D, D)], sem))
            for cp in copies:
                cp.wait()
            return 0

        lax.fori_loop(0, BPW // K, burst, 0)
        for cp in ucopies:
            cp.wait()

        w0 = w_v[0, :]
        w1 = w_v[1, :]
        w2 = w_v[2, :]
        w3 = w_v[3, :]
        bias = b_v[:]
        lanes = lax.iota(jnp.int32, 16)

        def block(blk, _):
            r0 = blk * 16
            res = jnp.zeros((16,), jnp.float32)
            for k in range(16):
                rr = r0 // 2 + k // 2
                c0 = (k % 2) * D
                uj = (r0 + k) // (BPW // 8)
                ur = (r0 + k) % (BPW // 8)
                t = (urows_v[uj, ur, pl.ds(0, 16)] * mrows_v[rr, pl.ds(c0, 16)] * w0
                     + urows_v[uj, ur, pl.ds(16, 16)] * mrows_v[rr, pl.ds(c0 + 16, 16)] * w1
                     + urows_v[uj, ur, pl.ds(32, 16)] * mrows_v[rr, pl.ds(c0 + 32, 16)] * w2
                     + urows_v[uj, ur, pl.ds(48, 16)] * mrows_v[rr, pl.ds(c0 + 48, 16)] * w3)
                s = jnp.sum(t + bias)
                res = jnp.where(lanes == k, s, res)
            out_v[pl.ds(r0, 16)] = res
            return 0

        lax.fori_loop(0, BPW // 16, block, 0)

        pltpu.sync_copy(out_v, out_hbm.at[pl.ds(base, BPW)])

    return body(movie_idx, u_g, movie_tab, w16, b16)


def kernel(user_idx, movie_idx, user_table, movie_table, fc_w, fc_b):
    uidx = user_idx.astype(jnp.int32)
    midx = movie_idx.astype(jnp.int32)
    w16 = fc_w.reshape(4, 16)
    b16 = jnp.broadcast_to(fc_b.reshape(1, 1) / 16.0, (1, 16)).reshape(16)

    perm = jnp.argsort(uidx)
    su = jnp.take(uidx, perm)
    pu = perm.astype(jnp.int32)
    utail = user_table[TAILSTART:, :]

    movie_rm = _tc_transpose(movie_table.T, movie_table.shape[0])
    u_g = _sc_user_stream(su, pu, user_table.T, utail)
    out = _sc_compute(midx, u_g, movie_rm, w16, b16)
    return out
